# baseline (speedup 1.0000x reference)
import functools

import jax
import jax.numpy as jnp
from jax import lax
from jax.experimental import pallas as pl
from jax.experimental.pallas import tpu as pltpu
from jax.experimental.pallas import tpu_sc as plsc

_NC = 2
_NS = 16
_NW = _NC * _NS
_L = 16
_CHUNK = 32
_NBUF = 3


def _dyn_gather(v, idx):
    return lax.gather(
        v, idx[:, None],
        lax.GatherDimensionNumbers(offset_dims=(), collapsed_slice_dims=(0,),
                                   start_index_map=(0,)),
        slice_sizes=(1,), mode=lax.GatherScatterMode.PROMISE_IN_BOUNDS)


def _bcast(v, j):
    return _dyn_gather(v, jnp.full((_L,), j, jnp.int32))


def _body(nseq, rows_per_w, d, x_hbm, cu_hbm, w_hbm, out_hbm,
          cu_vm, tok_vm, idx_vm, x_vm, w_vm, in_sems, out_sems, sem_w):
    cid = lax.axis_index("c")
    sid = lax.axis_index("s")
    wid = sid * _NC + cid
    base = wid * rows_per_w
    nchunks = rows_per_w // _CHUNK

    def w_stage_copies():
        return [pltpu.make_async_copy(w_hbm, w_vm.at[pl.ds(i, 1)], sem_w)
                for i in range(_L)]

    @pl.when(wid == 0)
    def _():
        for cp in w_stage_copies():
            cp.start()

    pltpu.sync_copy(cu_hbm, cu_vm.at[pl.ds(0, nseq + 1)])
    cu_v = cu_vm[...]
    bounds = [_bcast(cu_v, j) for j in range(1, nseq + 1)]

    def start_in(k):
        b = k % _NBUF
        return pltpu.async_copy(
            x_hbm.at[pl.ds(base + k * _CHUNK, _CHUNK)], x_vm[b], in_sems[b])

    ins = {k: start_in(k) for k in range(min(_NBUF, nchunks))}
    outs = {}
    for k in range(nchunks):
        b = k % _NBUF
        if k > 0 and (k - 1) + _NBUF < nchunks:
            outs.pop(k - 1).wait()
            ins[k - 1 + _NBUF] = start_in(k - 1 + _NBUF)
        start = base + k * _CHUNK
        first_pos = last_pos = None
        for g in range(_CHUNK // _L):
            t = start + g * _L + lax.iota(jnp.int32, _L)
            pos = t + 1
            for bound in bounds:
                pos = jnp.where(bound <= t, pos + 1, pos)
            idx_vm[b][pl.ds(g * _L, _L)] = pos
            if g == 0:
                first_pos = pos
            last_pos = pos
        dst0 = jnp.min(first_pos)
        contiguous = (jnp.max(last_pos) - dst0) == (_CHUNK - 1)
        ins[k].wait()

        @pl.when(contiguous)
        def _():
            pltpu.make_async_copy(
                x_vm[b], out_hbm.at[pl.ds(dst0, _CHUNK)],
                out_sems[b]).start()

        @pl.when(jnp.logical_not(contiguous))
        def _():
            pltpu.make_async_copy(
                x_vm[b], out_hbm.at[idx_vm[b]], out_sems[b]).start()

        outs[k] = pltpu.make_async_copy(
            x_vm[b], out_hbm.at[pl.ds(0, _CHUNK)], out_sems[b])
    for k in sorted(outs):
        outs.pop(k).wait()

    @pl.when(wid == 0)
    def _():
        m = jnp.bitwise_and(lax.iota(jnp.int32, _L), nseq - 1)
        tok_vm[...] = _dyn_gather(cu_v, m) + m
        for cp in w_stage_copies():
            cp.wait()
        pltpu.async_copy(w_vm, out_hbm.at[tok_vm], sem_w).wait()


def kernel(x_flat, cu_seqlens, weight):
    t_tok, d = x_flat.shape
    nseq = cu_seqlens.shape[0] - 1
    assert t_tok % (_NW * _CHUNK) == 0
    assert nseq & (nseq - 1) == 0 and nseq <= _L
    rows_per_w = t_tok // _NW

    mesh = plsc.VectorSubcoreMesh(core_axis_name="c", subcore_axis_name="s")
    run = pl.kernel(
        functools.partial(_body, nseq, rows_per_w, d),
        out_type=jax.ShapeDtypeStruct((t_tok + nseq, d), x_flat.dtype),
        mesh=mesh,
        compiler_params=pltpu.CompilerParams(use_tc_tiling_on_sc=False,
                                             needs_layout_passes=False),
        scratch_types=[
            pltpu.VMEM((_L,), jnp.int32),
            pltpu.VMEM((_L,), jnp.int32),
            [pltpu.VMEM((_CHUNK,), jnp.int32) for _ in range(_NBUF)],
            [pltpu.VMEM((_CHUNK, d), jnp.float32) for _ in range(_NBUF)],
            pltpu.VMEM((_L, d), jnp.float32),
            [pltpu.SemaphoreType.DMA for _ in range(_NBUF)],
            [pltpu.SemaphoreType.DMA for _ in range(_NBUF)],
            pltpu.SemaphoreType.DMA,
        ],
    )
    return run(x_flat, cu_seqlens.astype(jnp.int32), weight)


# speedup vs baseline: 2.3762x; 2.3762x over previous
import functools

import jax
import jax.numpy as jnp
from jax import lax
from jax.experimental import pallas as pl
from jax.experimental.pallas import tpu as pltpu
from jax.experimental.pallas import tpu_sc as plsc

_NC = 2
_NS = 16
_NW = _NC * _NS
_L = 16
_CHUNK = 16
_NBUF = 6
_OUTLAG = 4


def _dyn_gather(v, idx):
    return lax.gather(
        v, idx[:, None],
        lax.GatherDimensionNumbers(offset_dims=(), collapsed_slice_dims=(0,),
                                   start_index_map=(0,)),
        slice_sizes=(1,), mode=lax.GatherScatterMode.PROMISE_IN_BOUNDS)


def _bcast(v, j):
    return _dyn_gather(v, jnp.full((_L,), j, jnp.int32))


def _body(nseq, rows_per_w, d, x_hbm, cu_hbm, w_hbm, out_hbm,
          cu_vm, tok_vm, idx_vm, x_vm, w_vm, in_sems, out_sems, sem_w):
    cid = lax.axis_index("c")
    sid = lax.axis_index("s")
    wid = sid * _NC + cid
    base = wid * rows_per_w
    nchunks = rows_per_w // _CHUNK

    def w_stage_copies():
        return [pltpu.make_async_copy(w_hbm, w_vm.at[pl.ds(i, 1)], sem_w)
                for i in range(_L)]

    @pl.when(wid == 0)
    def _():
        for cp in w_stage_copies():
            cp.start()

    pltpu.sync_copy(cu_hbm, cu_vm.at[pl.ds(0, nseq + 1)])
    cu_v = cu_vm[...]
    bounds = [_bcast(cu_v, j) for j in range(1, nseq + 1)]

    def start_in(k):
        b = k % _NBUF
        return pltpu.async_copy(
            x_hbm.at[pl.ds(base + k * _CHUNK, _CHUNK)], x_vm[b], in_sems[b])

    ins = {k: start_in(k) for k in range(min(_NBUF, nchunks))}
    outs = {}
    for k in range(nchunks):
        b = k % _NBUF
        j = k - _OUTLAG
        if j >= 0 and j + _NBUF < nchunks:
            outs.pop(j).wait()
            ins[j + _NBUF] = start_in(j + _NBUF)
        start = base + k * _CHUNK
        for g in range(_CHUNK // _L):
            t = start + g * _L + lax.iota(jnp.int32, _L)
            pos = t + 1
            for bound in bounds:
                pos = jnp.where(bound <= t, pos + 1, pos)
            idx_vm[b][pl.ds(g * _L, _L)] = pos
        ins[k].wait()
        outs[k] = pltpu.async_copy(x_vm[b], out_hbm.at[idx_vm[b]],
                                   out_sems[b])
    for k in sorted(outs):
        outs.pop(k).wait()

    @pl.when(wid == 0)
    def _():
        m = jnp.bitwise_and(lax.iota(jnp.int32, _L), nseq - 1)
        tok_vm[...] = _dyn_gather(cu_v, m) + m
        for cp in w_stage_copies():
            cp.wait()
        pltpu.async_copy(w_vm, out_hbm.at[tok_vm], sem_w).wait()


def kernel(x_flat, cu_seqlens, weight):
    t_tok, d = x_flat.shape
    nseq = cu_seqlens.shape[0] - 1
    assert t_tok % (_NW * _CHUNK) == 0
    assert nseq & (nseq - 1) == 0 and nseq <= _L
    rows_per_w = t_tok // _NW

    mesh = plsc.VectorSubcoreMesh(core_axis_name="c", subcore_axis_name="s")
    run = pl.kernel(
        functools.partial(_body, nseq, rows_per_w, d),
        out_type=jax.ShapeDtypeStruct((t_tok + nseq, d), x_flat.dtype),
        mesh=mesh,
        scratch_types=[
            pltpu.VMEM((_L,), jnp.int32),
            pltpu.VMEM((_L,), jnp.int32),
            [pltpu.VMEM((_CHUNK,), jnp.int32) for _ in range(_NBUF)],
            [pltpu.VMEM((_CHUNK, d), jnp.float32) for _ in range(_NBUF)],
            pltpu.VMEM((_L, d), jnp.float32),
            [pltpu.SemaphoreType.DMA for _ in range(_NBUF)],
            [pltpu.SemaphoreType.DMA for _ in range(_NBUF)],
            pltpu.SemaphoreType.DMA,
        ],
    )
    return run(x_flat, cu_seqlens.astype(jnp.int32), weight)


# speedup vs baseline: 2.4586x; 1.0347x over previous
import functools

import jax
import jax.numpy as jnp
from jax import lax
from jax.experimental import pallas as pl
from jax.experimental.pallas import tpu as pltpu
from jax.experimental.pallas import tpu_sc as plsc

_NC = 2
_NS = 16
_NW = _NC * _NS
_L = 16
_CHUNK = 16
_NBUF = 6
_OUTLAG = 4


def _dyn_gather(v, idx):
    return lax.gather(
        v, idx[:, None],
        lax.GatherDimensionNumbers(offset_dims=(), collapsed_slice_dims=(0,),
                                   start_index_map=(0,)),
        slice_sizes=(1,), mode=lax.GatherScatterMode.PROMISE_IN_BOUNDS)


def _bcast(v, j):
    return _dyn_gather(v, jnp.full((_L,), j, jnp.int32))


def _body(nseq, rows_per_w, d, x_hbm, cu_hbm, w_hbm, out_hbm,
          cu_vm, tok_vm, idx_vm, x_vm, w_vm, in_sems, out_sems, sem_w):
    cid = lax.axis_index("c")
    sid = lax.axis_index("s")
    wid = sid * _NC + cid
    base = wid * rows_per_w
    nchunks = rows_per_w // _CHUNK

    def w_stage_copies():
        return [pltpu.make_async_copy(w_hbm, w_vm.at[pl.ds(i, 1)], sem_w)
                for i in range(_L)]

    @pl.when(wid == 1)
    def _():
        for cp in w_stage_copies():
            cp.start()

    pltpu.sync_copy(cu_hbm, cu_vm.at[pl.ds(0, nseq + 1)])
    cu_v = cu_vm[...]
    bounds = [_bcast(cu_v, j) for j in range(1, nseq + 1)]

    def start_in(k):
        b = k % _NBUF
        return pltpu.async_copy(
            x_hbm.at[pl.ds(base + k * _CHUNK, _CHUNK)], x_vm[b], in_sems[b])

    ins = {k: start_in(k) for k in range(min(_NBUF, nchunks))}
    outs = {}
    for k in range(nchunks):
        b = k % _NBUF
        j = k - _OUTLAG
        if j >= 0 and j + _NBUF < nchunks:
            outs.pop(j).wait()
            ins[j + _NBUF] = start_in(j + _NBUF)
        start = base + k * _CHUNK
        for g in range(_CHUNK // _L):
            t = start + g * _L + lax.iota(jnp.int32, _L)
            pos = t + 1
            for bound in bounds:
                pos = jnp.where(bound <= t, pos + 1, pos)
            idx_vm[b][pl.ds(g * _L, _L)] = pos
        ins[k].wait()
        outs[k] = pltpu.async_copy(x_vm[b], out_hbm.at[idx_vm[b]],
                                   out_sems[b])
        if k == nchunks // 2:
            @pl.when(wid == 1)
            def _():
                m = jnp.bitwise_and(lax.iota(jnp.int32, _L), nseq - 1)
                tok_vm[...] = _dyn_gather(cu_v, m) + m
                for cp in w_stage_copies():
                    cp.wait()
                pltpu.make_async_copy(w_vm, out_hbm.at[tok_vm],
                                      sem_w).start()
    for k in sorted(outs):
        outs.pop(k).wait()

    @pl.when(wid == 1)
    def _():
        pltpu.make_async_copy(w_vm, out_hbm.at[tok_vm], sem_w).wait()


def kernel(x_flat, cu_seqlens, weight):
    t_tok, d = x_flat.shape
    nseq = cu_seqlens.shape[0] - 1
    assert t_tok % (_NW * _CHUNK) == 0
    assert nseq & (nseq - 1) == 0 and nseq <= _L
    rows_per_w = t_tok // _NW

    mesh = plsc.VectorSubcoreMesh(core_axis_name="c", subcore_axis_name="s")
    run = pl.kernel(
        functools.partial(_body, nseq, rows_per_w, d),
        out_type=jax.ShapeDtypeStruct((t_tok + nseq, d), x_flat.dtype),
        mesh=mesh,
        scratch_types=[
            pltpu.VMEM((_L,), jnp.int32),
            pltpu.VMEM((_L,), jnp.int32),
            [pltpu.VMEM((_CHUNK,), jnp.int32) for _ in range(_NBUF)],
            [pltpu.VMEM((_CHUNK, d), jnp.float32) for _ in range(_NBUF)],
            pltpu.VMEM((_L, d), jnp.float32),
            [pltpu.SemaphoreType.DMA for _ in range(_NBUF)],
            [pltpu.SemaphoreType.DMA for _ in range(_NBUF)],
            pltpu.SemaphoreType.DMA,
        ],
    )
    return run(x_flat, cu_seqlens.astype(jnp.int32), weight)
